# baseline (speedup 1.0000x reference)
import functools

import jax
import jax.numpy as jnp
from jax import lax
from jax.experimental import pallas as pl
from jax.experimental.pallas import tpu as pltpu
from jax.experimental.pallas import tpu_sc as plsc

_MAX_DIST = 128
_H = 16
_S = 2048
_TAB = 2 * _MAX_DIST + 1
_EPAD = 264
_R = 16
_W = 4096
_UNROLL = 8
_PIPE = 4


def _sc_body(emb_t, out, e_row, w, sem):
    wid = lax.axis_index("s") * 2 + lax.axis_index("c")
    head = wid >> 1
    i0 = (wid & 1) * (_S // 2)

    pltpu.sync_copy(emb_t.at[head], e_row)

    iota = lax.iota(jnp.int32, 16)

    for r in range(_R):
        def build(c, _, r=r):
            for u in range(_UNROLL):
                base = (c * _UNROLL + u) * 16
                t = base + iota + (7 - r - (_S - 1))
                idx = jnp.clip(t, -_MAX_DIST, _MAX_DIST) + _MAX_DIST
                w[r, pl.ds(base, 16)] = plsc.load_gather(e_row, [idx])
            return 0
        lax.fori_loop(0, _W // (16 * _UNROLL), build, 0)

    def block(b, _):
        i = i0 + b * _R
        k0 = pl.multiple_of((_S - 8) - i, 8)
        pltpu.make_async_copy(
            w.at[:, pl.ds(k0, _S)], out.at[head, pl.ds(i, _R), :], sem
        ).start()

        @pl.when(b >= _PIPE)
        def _():
            pltpu.make_async_copy(
                w.at[:, pl.ds(0, _S)], out.at[head, pl.ds(i0, _R), :], sem
            ).wait()

        return 0

    lax.fori_loop(0, _S // 2 // _R, block, 0)
    for _ in range(_PIPE):
        pltpu.make_async_copy(
            w.at[:, pl.ds(0, _S)], out.at[head, pl.ds(i0, _R), :], sem
        ).wait()


@functools.partial(jax.jit)
def _rpb_sc(emb_t):
    mesh = plsc.VectorSubcoreMesh(core_axis_name="c", subcore_axis_name="s")
    f = functools.partial(
        pl.kernel,
        mesh=mesh,
        out_type=jax.ShapeDtypeStruct((_H, _S, _S), jnp.float32),
        compiler_params=pltpu.CompilerParams(
            needs_layout_passes=False, use_tc_tiling_on_sc=False
        ),
        scratch_types=[
            pltpu.VMEM((_EPAD,), jnp.float32),
            pltpu.VMEM((_R, _W), jnp.float32),
            pltpu.SemaphoreType.DMA,
        ],
    )(_sc_body)
    return f(emb_t)


def kernel(seq_len, embeddings):
    del seq_len
    emb_t = jnp.zeros((_H, _EPAD), jnp.float32).at[:, :_TAB].set(embeddings.T)
    return _rpb_sc(emb_t)


# speedup vs baseline: 2.8856x; 2.8856x over previous
import functools

import jax
import jax.numpy as jnp
from jax import lax
from jax.experimental import pallas as pl
from jax.experimental.pallas import tpu as pltpu
from jax.experimental.pallas import tpu_sc as plsc

_MAX_DIST = 128
_H = 16
_S = 2048
_TAB = 2 * _MAX_DIST + 1
_EPAD = 264
_WLEN = 4096
_UNROLL = 8
_FILL0_CHUNKS = 112
_BAND_CHUNKS = 24
_FILL1_CHUNKS = 120


def _build_w(w, b, e_row, e0_vec, e256_vec, shift):
    iota = lax.iota(jnp.int32, 16)
    for r in range(8):
        def fill0(q, _, r=r):
            for u in range(8):
                w[b, q, r, pl.ds(u * 16, 16)] = e0_vec
            return 0
        lax.fori_loop(0, 14, fill0, 0)

        def band(qq, _, r=r):
            q = qq + 14
            for u in range(8):
                k = q * 128 + u * 16
                t = k + iota + (shift - r - (_S - 1))
                idx = jnp.clip(t, -_MAX_DIST, _MAX_DIST) + _MAX_DIST
                w[b, q, r, pl.ds(u * 16, 16)] = plsc.load_gather(e_row, [idx])
            return 0
        lax.fori_loop(0, 3, band, 0)

        def fill1(qq, _, r=r):
            q = qq + 17
            for u in range(8):
                w[b, q, r, pl.ds(u * 16, 16)] = e256_vec
            return 0
        lax.fori_loop(0, 15, fill1, 0)


def _sc_body(emb_flat, out, e_row, w, sem0, sem1):
    wid = lax.axis_index("s") * 2 + lax.axis_index("c")
    m = wid & 15
    h_base = (wid >> 4) * 8
    shift = (120 - 8 * m) + 7

    sems = (sem0, sem1)

    def drain_buffer(sem):
        def one(_i, _):
            pltpu.make_async_copy(
                w.at[0, 0],
                out.at[0, pl.ds(0, 8), pl.ds(0, 128)],
                sem,
            ).wait()
            return 0

        lax.fori_loop(0, 256, one, 0)

    def head_work(h, b, sem):
        pltpu.sync_copy(emb_flat.at[pl.ds(h * _EPAD, _EPAD)], e_row)
        lo = e_row[pl.ds(0, 16)]
        hi = e_row[pl.ds(_TAB - 9, 16)]
        e0_vec = jnp.full((16,), lo[0], jnp.float32)
        e256_vec = jnp.full((16,), hi[8], jnp.float32)

        _build_w(w, b, e_row, e0_vec, e256_vec, shift)

        def tile_dma(pt, _):
            p = pt >> 4
            t = pt & 15
            q = 15 - p + t
            i = pl.multiple_of(8 * m + 128 * p, 8)
            j = pl.multiple_of(128 * t, 128)
            pltpu.make_async_copy(
                w.at[b, q],
                out.at[h, pl.ds(i, 8), pl.ds(j, 128)],
                sem,
            ).start()
            return 0

        lax.fori_loop(0, 256, tile_dma, 0)

    head_work(h_base + 0, 0, sem0)
    head_work(h_base + 1, 1, sem1)

    def pair_step(step, _):
        h = h_base + 2 * (step + 1)
        drain_buffer(sem0)
        head_work(h, 0, sem0)
        drain_buffer(sem1)
        head_work(h + 1, 1, sem1)
        return 0

    lax.fori_loop(0, 3, pair_step, 0)

    drain_buffer(sem0)
    drain_buffer(sem1)


@functools.partial(jax.jit)
def _rpb_sc(emb_flat):
    mesh = plsc.VectorSubcoreMesh(core_axis_name="c", subcore_axis_name="s")
    f = functools.partial(
        pl.kernel,
        mesh=mesh,
        out_type=jax.ShapeDtypeStruct((_H, _S, _S), jnp.float32),
        compiler_params=pltpu.CompilerParams(needs_layout_passes=False),
        scratch_types=[
            pltpu.VMEM((_EPAD,), jnp.float32),
            pltpu.VMEM((2, 32, 8, 128), jnp.float32),
            pltpu.SemaphoreType.DMA,
            pltpu.SemaphoreType.DMA,
        ],
    )(_sc_body)
    return f(emb_flat)


def kernel(seq_len, embeddings):
    del seq_len
    emb_t = jnp.zeros((_H, _EPAD), jnp.float32).at[:, :_TAB].set(embeddings.T)
    return _rpb_sc(emb_t.reshape(-1))


# speedup vs baseline: 3.1235x; 1.0824x over previous
import functools

import jax
import jax.numpy as jnp
from jax import lax
from jax.experimental import pallas as pl
from jax.experimental.pallas import tpu as pltpu
from jax.experimental.pallas import tpu_sc as plsc

_MAX_DIST = 128
_H = 16
_S = 2048
_TAB = 2 * _MAX_DIST + 1
_EPAD = 264
_WLEN = 4096
_UNROLL = 8
_FILL0_CHUNKS = 112
_BAND_CHUNKS = 24
_FILL1_CHUNKS = 120


def _build_w(w, b, e_row, e0_vec, e256_vec, shift):
    iota = lax.iota(jnp.int32, 16)
    for r in range(8):
        for u in range(8):
            w[b, 0, r, pl.ds(u * 16, 16)] = e0_vec
            w[b, 4, r, pl.ds(u * 16, 16)] = e256_vec

        def band(qq, _, r=r):
            for u in range(8):
                k = (qq + 14) * 128 + u * 16
                t = k + iota + (shift - r - (_S - 1))
                idx = jnp.clip(t, -_MAX_DIST, _MAX_DIST) + _MAX_DIST
                w[b, qq + 1, r, pl.ds(u * 16, 16)] = plsc.load_gather(
                    e_row, [idx]
                )
            return 0
        lax.fori_loop(0, 3, band, 0)


def _sc_body(emb_flat, out, e_row, w, sem0, sem1):
    wid = lax.axis_index("s") * 2 + lax.axis_index("c")
    m = wid & 15
    h_base = (wid >> 4) * 8
    shift = (120 - 8 * m) + 7

    sems = (sem0, sem1)

    def drain_buffer(sem):
        def one(_i, _):
            pltpu.make_async_copy(
                w.at[0, 0],
                out.at[0, pl.ds(0, 8), pl.ds(0, 128)],
                sem,
            ).wait()
            return 0

        lax.fori_loop(0, 256, one, 0)

    def head_work(h, b, sem):
        pltpu.sync_copy(emb_flat.at[pl.ds(h * _EPAD, _EPAD)], e_row)
        lo = e_row[pl.ds(0, 16)]
        hi = e_row[pl.ds(_TAB - 9, 16)]
        e0_vec = jnp.full((16,), lo[0], jnp.float32)
        e256_vec = jnp.full((16,), hi[8], jnp.float32)

        _build_w(w, b, e_row, e0_vec, e256_vec, shift)

        def tile_dma(pt, _):
            p = pt >> 4
            t = pt & 15
            qsel = jnp.clip((15 - p + t) - 13, 0, 4)
            i = pl.multiple_of(8 * m + 128 * p, 8)
            j = pl.multiple_of(128 * t, 128)
            pltpu.make_async_copy(
                w.at[b, qsel],
                out.at[h, pl.ds(i, 8), pl.ds(j, 128)],
                sem,
            ).start()
            return 0

        lax.fori_loop(0, 256, tile_dma, 0)

    head_work(h_base + 0, 0, sem0)
    head_work(h_base + 1, 1, sem1)

    def pair_step(step, _):
        h = h_base + 2 * (step + 1)
        drain_buffer(sem0)
        head_work(h, 0, sem0)
        drain_buffer(sem1)
        head_work(h + 1, 1, sem1)
        return 0

    lax.fori_loop(0, 3, pair_step, 0)

    drain_buffer(sem0)
    drain_buffer(sem1)


@functools.partial(jax.jit)
def _rpb_sc(emb_flat):
    mesh = plsc.VectorSubcoreMesh(core_axis_name="c", subcore_axis_name="s")
    f = functools.partial(
        pl.kernel,
        mesh=mesh,
        out_type=jax.ShapeDtypeStruct((_H, _S, _S), jnp.float32),
        compiler_params=pltpu.CompilerParams(needs_layout_passes=False),
        scratch_types=[
            pltpu.VMEM((_EPAD,), jnp.float32),
            pltpu.VMEM((2, 5, 8, 128), jnp.float32),
            pltpu.SemaphoreType.DMA,
            pltpu.SemaphoreType.DMA,
        ],
    )(_sc_body)
    return f(emb_flat)


def kernel(seq_len, embeddings):
    del seq_len
    emb_t = jnp.zeros((_H, _EPAD), jnp.float32).at[:, :_TAB].set(embeddings.T)
    return _rpb_sc(emb_t.reshape(-1))


# speedup vs baseline: 3.1573x; 1.0108x over previous
import functools

import jax
import jax.numpy as jnp
from jax import lax
from jax.experimental import pallas as pl
from jax.experimental.pallas import tpu as pltpu
from jax.experimental.pallas import tpu_sc as plsc

_MAX_DIST = 128
_H = 16
_S = 2048
_TAB = 2 * _MAX_DIST + 1
_EPAD = 264
_WLEN = 4096
_BAND_LO = 14
_BAND_HI = 17


def _build_w(w, b, e_row, e0_vec, e256_vec, shift):
    iota = lax.iota(jnp.int32, 16)
    for r in range(8):
        def fill0(c, _, r=r):
            for u in range(8):
                w[b, r, pl.ds((c * 8 + u) * 16, 16)] = e0_vec
            return 0
        lax.fori_loop(0, _BAND_LO, fill0, 0)

        def band(qq, _, r=r):
            for u in range(8):
                k = (_BAND_LO + qq) * 128 + u * 16
                t = k + iota + (shift - r - (_S - 1))
                idx = jnp.clip(t, -_MAX_DIST, _MAX_DIST) + _MAX_DIST
                w[b, r, pl.ds(k, 16)] = plsc.load_gather(e_row, [idx])
            return 0
        lax.fori_loop(0, _BAND_HI - _BAND_LO, band, 0)

        def fill1(c, _, r=r):
            for u in range(8):
                w[b, r, pl.ds(_BAND_HI * 128 + (c * 8 + u) * 16, 16)] = e256_vec
            return 0
        lax.fori_loop(0, 32 - _BAND_HI, fill1, 0)


def _sc_body(emb_flat, out, e_row, w, sem0, sem1):
    wid = lax.axis_index("s") * 2 + lax.axis_index("c")
    m = wid & 15
    h_base = (wid >> 4) * 8
    shift = (120 - 8 * m) + 7

    sems = (sem0, sem1)

    def drain_buffer(sem):
        def one(_i, _):
            pltpu.make_async_copy(
                w.at[0, :, pl.ds(0, _S)],
                out.at[0, pl.ds(0, 8), pl.ds(0, _S)],
                sem,
            ).wait()
            return 0

        lax.fori_loop(0, 16, one, 0)

    def head_work(h, b, sem):
        pltpu.sync_copy(emb_flat.at[pl.ds(h * _EPAD, _EPAD)], e_row)
        lo = e_row[pl.ds(0, 16)]
        hi = e_row[pl.ds(_TAB - 9, 16)]
        e0_vec = jnp.full((16,), lo[0], jnp.float32)
        e256_vec = jnp.full((16,), hi[8], jnp.float32)

        _build_w(w, b, e_row, e0_vec, e256_vec, shift)

        for p in range(16):
            i = pl.multiple_of(8 * m + 128 * p, 8)

            def start(koff, j, width, sem=sem, i=i, b=b, h=h):
                pltpu.make_async_copy(
                    w.at[b, :, pl.ds(koff, width)],
                    out.at[h, pl.ds(i, 8), pl.ds(j, width)],
                    sem,
                ).start()

            if p >= 2:
                start(128 * (15 - p), 0, 128 * (p - 1))
            for t in (p - 1, p, p + 1):
                if 0 <= t < 16:
                    start(128 * (15 - p + t), 128 * t, 128)
            if p <= 13:
                start(128 * 17, 128 * (p + 2), 128 * (14 - p))

    head_work(h_base + 0, 0, sem0)
    head_work(h_base + 1, 1, sem1)

    def pair_step(step, _):
        h = h_base + 2 * (step + 1)
        drain_buffer(sem0)
        head_work(h, 0, sem0)
        drain_buffer(sem1)
        head_work(h + 1, 1, sem1)
        return 0

    lax.fori_loop(0, 3, pair_step, 0)

    drain_buffer(sem0)
    drain_buffer(sem1)


@functools.partial(jax.jit)
def _rpb_sc(emb_flat):
    mesh = plsc.VectorSubcoreMesh(core_axis_name="c", subcore_axis_name="s")
    f = functools.partial(
        pl.kernel,
        mesh=mesh,
        out_type=jax.ShapeDtypeStruct((_H, _S, _S), jnp.float32),
        compiler_params=pltpu.CompilerParams(needs_layout_passes=False),
        scratch_types=[
            pltpu.VMEM((_EPAD,), jnp.float32),
            pltpu.VMEM((2, 8, _WLEN), jnp.float32),
            pltpu.SemaphoreType.DMA,
            pltpu.SemaphoreType.DMA,
        ],
    )(_sc_body)
    return f(emb_flat)


def kernel(seq_len, embeddings):
    del seq_len
    emb_t = jnp.zeros((_H, _EPAD), jnp.float32).at[:, :_TAB].set(embeddings.T)
    return _rpb_sc(emb_t.reshape(-1))
